# baseline (speedup 1.0000x reference)
import functools

import jax
import jax.numpy as jnp
from jax import lax
from jax.experimental import pallas as pl
from jax.experimental.pallas import tpu as pltpu
from jax.experimental.pallas import tpu_sc as plsc

VOCAB = 1000000
EMB = 64
BATCH = 16384
NEGS = 15

NC = 2
NS = 16
L = 16
NW = NC * NS
BPW = BATCH // NW
GSZ = 16
G = BPW // GSZ
NSLOT = 1 + NEGS
NEG_G = NSLOT * GSZ
NCH = EMB // L
NPW = BPW * NSLOT
PR = 2 * EMB


def _bcast_lane(v, i):
    idx = jnp.full((L, 1), i, dtype=jnp.int32)
    return lax.gather(
        v, idx,
        lax.GatherDimensionNumbers(offset_dims=(), collapsed_slice_dims=(0,),
                                   start_index_map=(0,)),
        slice_sizes=(1,), mode=lax.GatherScatterMode.PROMISE_IN_BOUNDS)


def _sc_dots_kernel(center_hbm, context_hbm, noise_hbm, mask_hbm,
                    wc_hbm, wx_hbm, out_hbm,
                    cst_v, xst_v, nst_v, cpi_v, xpi_v, npi_v,
                    cpb_v, xpb_v, npb_v,
                    cen0, ctx0, msk0, neg0, cen1, ctx1, msk1, neg1,
                    dots_v, sem0, sem1):
    wid = lax.axis_index("s") * NC + lax.axis_index("c")
    base = pl.multiple_of(wid * BPW, BPW)

    pltpu.sync_copy(center_hbm.at[wid], cst_v)
    pltpu.sync_copy(context_hbm.at[wid], xst_v)
    pltpu.sync_copy(noise_hbm.at[wid], nst_v)

    one = jnp.int32(1)

    def c_tr(i, carry):
        r = i >> 3
        c = pl.multiple_of((i & 7) * L, L)
        f = pl.multiple_of(i * L, L)
        cv = cst_v[r, pl.ds(c, L)]
        xv = xst_v[r, pl.ds(c, L)]
        cpi_v[pl.ds(f, L)] = lax.shift_right_logical(cv, 1)
        xpi_v[pl.ds(f, L)] = lax.shift_right_logical(xv, 1)
        cpb_v[pl.ds(f, L)] = cv & one
        xpb_v[pl.ds(f, L)] = xv & one
        return carry

    lax.fori_loop(0, BPW // L, c_tr, 0)

    def n_tr(i, carry):
        r = i >> 3
        c = pl.multiple_of((i & 7) * L, L)
        f = pl.multiple_of(i * L, L)
        nv = nst_v[r, pl.ds(c, L)]
        npi_v[pl.ds(f, L)] = lax.shift_right_logical(nv, 1)
        npb_v[pl.ds(f, L)] = nv & one
        return carry

    lax.fori_loop(0, NPW // L, n_tr, 0)

    bufs = ((cen0, ctx0, msk0, neg0, sem0), (cen1, ctx1, msk1, neg1, sem1))

    def issue(g, s):
        cen_v, ctx_v, msk_v, neg_v, sem = bufs[s]
        row0 = pl.multiple_of(g * GSZ, GSZ)
        pltpu.async_copy(wc_hbm.at[cpi_v.at[pl.ds(row0, GSZ)]], cen_v, sem)
        pltpu.async_copy(wx_hbm.at[xpi_v.at[pl.ds(row0, GSZ)]], ctx_v, sem)
        pltpu.async_copy(wx_hbm.at[npi_v.at[pl.ds(row0 * NSLOT, NEG_G)]],
                         neg_v, sem)
        mrow0 = pl.multiple_of((base + row0) // 2, GSZ // 2)
        pltpu.async_copy(mask_hbm.at[pl.ds(mrow0, GSZ // 2)], msk_v, sem)

    def drain(s):
        cen_v, ctx_v, msk_v, neg_v, sem = bufs[s]
        pltpu.make_async_copy(wc_hbm.at[pl.ds(0, GSZ)], cen_v, sem).wait()
        pltpu.make_async_copy(wx_hbm.at[pl.ds(0, GSZ)], ctx_v, sem).wait()
        pltpu.make_async_copy(wx_hbm.at[pl.ds(0, NEG_G)], neg_v, sem).wait()
        pltpu.make_async_copy(mask_hbm.at[pl.ds(0, GSZ // 2)], msk_v,
                              sem).wait()

    lane = lax.iota(jnp.int32, L)
    lane_mask = [lane == k for k in range(L)]

    def halves(buf, row, sel):
        return [jnp.where(sel,
                          buf[row, pl.ds(EMB + c * L, L)],
                          buf[row, pl.ds(c * L, L)])
                for c in range(NCH)]

    def compute(g, s):
        cen_v, ctx_v, msk_v, neg_v, _ = bufs[s]
        row0 = pl.multiple_of(g * GSZ, GSZ)
        cpo = cpb_v[pl.ds(row0, GSZ)]
        xpo = xpb_v[pl.ds(row0, GSZ)]

        def body(b, carry):
            q = row0 + b
            selc = _bcast_lane(cpo, b) == one
            selx = _bcast_lane(xpo, b) == one
            hm = pl.multiple_of((b & 1) * EMB, EMB)
            ch = halves(cen_v, b, selc)
            cm = [ch[c] * msk_v[b >> 1, pl.ds(hm + c * L, L)]
                  for c in range(NCH)]
            xh = halves(ctx_v, b, selx)
            t = cm[0] * xh[0]
            for c in range(1, NCH):
                t = t + cm[c] * xh[c]
            acc = jnp.where(lane_mask[0], jnp.sum(t), 0.0)
            npo = npb_v[pl.ds(pl.multiple_of(q * NSLOT, L), L)]
            for k in range(1, NSLOT):
                seln = _bcast_lane(npo, k) == one
                nh = halves(neg_v, b * NSLOT + k, seln)
                t = cm[0] * nh[0]
                for c in range(1, NCH):
                    t = t + cm[c] * nh[c]
                acc = jnp.where(lane_mask[k], jnp.sum(t), acc)
            f = q * L
            dots_v[f >> 7, pl.ds(pl.multiple_of(f & 127, L), L)] = acc
            return carry

        lax.fori_loop(0, GSZ, body, 0)

    issue(0, 0)

    def pair(i, carry):
        g0 = pl.multiple_of(i * 2, 2)
        issue(g0 + 1, 1)
        drain(0)
        compute(g0, 0)

        @pl.when(g0 + 2 < G)
        def _():
            issue(g0 + 2, 0)

        drain(1)
        compute(g0 + 1, 1)
        return carry

    lax.fori_loop(0, G // 2, pair, 0)
    pltpu.sync_copy(dots_v, out_hbm.at[wid])


@functools.partial(
    pl.kernel,
    out_type=jax.ShapeDtypeStruct((NW, BPW * NSLOT // PR, PR), jnp.float32),
    mesh=plsc.VectorSubcoreMesh(core_axis_name="c", subcore_axis_name="s"),
    compiler_params=pltpu.CompilerParams(needs_layout_passes=False,
                                         use_tc_tiling_on_sc=True),
    scratch_types=[
        pltpu.VMEM((BPW // PR, PR), jnp.int32),
        pltpu.VMEM((BPW // PR, PR), jnp.int32),
        pltpu.VMEM((NPW // PR, PR), jnp.int32),
        pltpu.VMEM((BPW,), jnp.int32),
        pltpu.VMEM((BPW,), jnp.int32),
        pltpu.VMEM((NPW,), jnp.int32),
        pltpu.VMEM((BPW,), jnp.int32),
        pltpu.VMEM((BPW,), jnp.int32),
        pltpu.VMEM((NPW,), jnp.int32),
        pltpu.VMEM((GSZ, PR), jnp.float32),
        pltpu.VMEM((GSZ, PR), jnp.float32),
        pltpu.VMEM((GSZ // 2, PR), jnp.float32),
        pltpu.VMEM((NEG_G, PR), jnp.float32),
        pltpu.VMEM((GSZ, PR), jnp.float32),
        pltpu.VMEM((GSZ, PR), jnp.float32),
        pltpu.VMEM((GSZ // 2, PR), jnp.float32),
        pltpu.VMEM((NEG_G, PR), jnp.float32),
        pltpu.VMEM((BPW * NSLOT // PR, PR), jnp.float32),
        pltpu.SemaphoreType.DMA,
        pltpu.SemaphoreType.DMA,
    ],
)
def _sc_dots(center, context, noise, mask, wc, wx, out,
             cst_v, xst_v, nst_v, cpi_v, xpi_v, npi_v, cpb_v, xpb_v, npb_v,
             cen0, ctx0, msk0, neg0, cen1, ctx1, msk1, neg1,
             dots_v, sem0, sem1):
    _sc_dots_kernel(center, context, noise, mask, wc, wx, out,
                    cst_v, xst_v, nst_v, cpi_v, xpi_v, npi_v,
                    cpb_v, xpb_v, npb_v,
                    cen0, ctx0, msk0, neg0, cen1, ctx1, msk1, neg1,
                    dots_v, sem0, sem1)


VB = 2048


def _tc_relayout_kernel(wt_ref, o_ref):
    t = jnp.transpose(wt_ref[...], (1, 0)).reshape(VB // 2, 2, EMB)
    o_ref[:, 0:EMB] = t[:, 0, :]
    o_ref[:, EMB:PR] = t[:, 1, :]


def _tc_relayout(wt):
    return pl.pallas_call(
        _tc_relayout_kernel,
        grid=(VOCAB // VB,),
        in_specs=[pl.BlockSpec((EMB, VB), lambda j: (0, j))],
        out_specs=pl.BlockSpec((VB // 2, PR), lambda j: (j, 0)),
        out_shape=jax.ShapeDtypeStruct((VOCAB // 2, PR), jnp.float32),
    )(wt)


def _tc_loss_kernel(dots_ref, o_ref):
    x = dots_ref[...]
    col = lax.broadcasted_iota(jnp.int32, x.shape, 2)
    z = jnp.where((col & (L - 1)) == 0, x, -x)
    ls = jnp.minimum(z, 0.0) - jnp.log1p(jnp.exp(-jnp.abs(z)))
    o_ref[0, 0] = -jnp.sum(ls) / jnp.float32(BATCH)


def kernel(center, context, W_center, W_context):
    dk = jax.random.key(123)
    keep = jax.random.bernoulli(jax.random.fold_in(dk, 0), 0.9, (BATCH, EMB))
    mask_scale = jnp.where(keep, jnp.float32(1.0 / 0.9), jnp.float32(0.0))
    noise_idx = jax.random.randint(jax.random.fold_in(dk, 1), (BATCH * NEGS,),
                                   0, VOCAB).astype(jnp.int32)
    noise15 = noise_idx.reshape(BATCH, NEGS)
    noise16 = jnp.concatenate([noise15[:, :1], noise15], axis=1)

    dots = _sc_dots(
        center.astype(jnp.int32).reshape(NW, BPW // PR, PR),
        context.astype(jnp.int32).reshape(NW, BPW // PR, PR),
        noise16.reshape(NW, NPW // PR, PR),
        mask_scale.reshape(BATCH * EMB // PR, PR),
        _tc_relayout(W_center.T),
        _tc_relayout(W_context.T),
    )

    loss = pl.pallas_call(
        _tc_loss_kernel,
        out_shape=jax.ShapeDtypeStruct((1, 1), jnp.float32),
        out_specs=pl.BlockSpec(memory_space=pltpu.SMEM),
    )(dots)
    return loss[0, 0]


# speedup vs baseline: 1.1325x; 1.1325x over previous
import functools

import jax
import jax.numpy as jnp
from jax import lax
from jax.experimental import pallas as pl
from jax.experimental.pallas import tpu as pltpu
from jax.experimental.pallas import tpu_sc as plsc

VOCAB = 1000000
EMB = 64
BATCH = 16384
NEGS = 15

NC = 2
NS = 16
L = 16
NW = NC * NS
BPW = BATCH // NW
GSZ = 32
G = BPW // GSZ
NEG_G = NEGS * GSZ
NCH = EMB // L


def _sc_dots_kernel(center_hbm, context_hbm, noise_hbm, mask_hbm,
                    wc_hbm, wx_hbm, out_hbm,
                    cidx_v, xidx_v, nidx_v,
                    cen0, ctx0, msk0, neg0, cen1, ctx1, msk1, neg1,
                    dots_v, sem0, sem1):
    wid = lax.axis_index("s") * NC + lax.axis_index("c")
    base = pl.multiple_of(wid * BPW, BPW)

    pltpu.sync_copy(center_hbm.at[pl.ds(base, BPW)], cidx_v)
    pltpu.sync_copy(context_hbm.at[pl.ds(base, BPW)], xidx_v)
    pltpu.sync_copy(noise_hbm.at[wid], nidx_v)

    bufs = ((cen0, ctx0, msk0, neg0, sem0), (cen1, ctx1, msk1, neg1, sem1))

    def issue(g, s):
        cen_v, ctx_v, msk_v, neg_v, sem = bufs[s]
        row0 = pl.multiple_of(g * GSZ, GSZ)
        pltpu.async_copy(wc_hbm.at[cidx_v.at[pl.ds(row0, GSZ)]], cen_v, sem)
        pltpu.async_copy(wx_hbm.at[xidx_v.at[pl.ds(row0, GSZ)]], ctx_v, sem)
        pltpu.async_copy(wx_hbm.at[nidx_v.at[g]], neg_v, sem)
        pltpu.async_copy(mask_hbm.at[pl.ds(base + row0, GSZ)], msk_v, sem)

    def drain(s):
        cen_v, ctx_v, msk_v, neg_v, sem = bufs[s]
        pltpu.make_async_copy(wc_hbm.at[pl.ds(0, GSZ)], cen_v, sem).wait()
        pltpu.make_async_copy(wx_hbm.at[pl.ds(0, GSZ)], ctx_v, sem).wait()
        pltpu.make_async_copy(wx_hbm.at[pl.ds(0, NEG_G)], neg_v, sem).wait()
        pltpu.make_async_copy(mask_hbm.at[pl.ds(0, GSZ)], msk_v, sem).wait()

    lane = lax.iota(jnp.int32, L)
    lane_mask = [lane == k for k in range(L)]

    def compute(g, s):
        cen_v, ctx_v, msk_v, neg_v, _ = bufs[s]
        row0 = pl.multiple_of(g * GSZ, GSZ)

        def body(b, carry):
            cm = [cen_v[b, pl.ds(c * L, L)] * msk_v[b, pl.ds(c * L, L)]
                  for c in range(NCH)]
            t = cm[0] * ctx_v[b, pl.ds(0, L)]
            for c in range(1, NCH):
                t = t + cm[c] * ctx_v[b, pl.ds(c * L, L)]
            acc = jnp.where(lane_mask[0], jnp.sum(t), 0.0)
            for j in range(NEGS):
                nrow = b * NEGS + j
                t = cm[0] * neg_v[nrow, pl.ds(0, L)]
                for c in range(1, NCH):
                    t = t + cm[c] * neg_v[nrow, pl.ds(c * L, L)]
                acc = jnp.where(lane_mask[1 + j], jnp.sum(t), acc)
            dots_v[row0 + b, :] = acc
            return carry

        lax.fori_loop(0, GSZ, body, 0)

    issue(0, 0)

    def pair(i, carry):
        g0 = pl.multiple_of(i * 2, 2)
        issue(g0 + 1, 1)
        drain(0)
        compute(g0, 0)

        @pl.when(g0 + 2 < G)
        def _():
            issue(g0 + 2, 0)

        drain(1)
        compute(g0 + 1, 1)
        return carry

    lax.fori_loop(0, G // 2, pair, 0)
    pltpu.sync_copy(dots_v, out_hbm.at[pl.ds(base, BPW)])


@functools.partial(
    pl.kernel,
    out_type=jax.ShapeDtypeStruct((BATCH, 1 + NEGS), jnp.float32),
    mesh=plsc.VectorSubcoreMesh(core_axis_name="c", subcore_axis_name="s"),
    compiler_params=pltpu.CompilerParams(needs_layout_passes=False,
                                         use_tc_tiling_on_sc=False),
    scratch_types=[
        pltpu.VMEM((BPW,), jnp.int32),
        pltpu.VMEM((BPW,), jnp.int32),
        pltpu.VMEM((G, NEG_G), jnp.int32),
        pltpu.VMEM((GSZ, EMB), jnp.float32),
        pltpu.VMEM((GSZ, EMB), jnp.float32),
        pltpu.VMEM((GSZ, EMB), jnp.float32),
        pltpu.VMEM((NEG_G, EMB), jnp.float32),
        pltpu.VMEM((GSZ, EMB), jnp.float32),
        pltpu.VMEM((GSZ, EMB), jnp.float32),
        pltpu.VMEM((GSZ, EMB), jnp.float32),
        pltpu.VMEM((NEG_G, EMB), jnp.float32),
        pltpu.VMEM((BPW, 1 + NEGS), jnp.float32),
        pltpu.SemaphoreType.DMA,
        pltpu.SemaphoreType.DMA,
    ],
)
def _sc_dots(center, context, noise, mask, wc, wx, out,
             cidx_v, xidx_v, nidx_v,
             cen0, ctx0, msk0, neg0, cen1, ctx1, msk1, neg1,
             dots_v, sem0, sem1):
    _sc_dots_kernel(center, context, noise, mask, wc, wx, out,
                    cidx_v, xidx_v, nidx_v,
                    cen0, ctx0, msk0, neg0, cen1, ctx1, msk1, neg1,
                    dots_v, sem0, sem1)


def _tc_loss_kernel(dots_ref, o_ref):
    x = dots_ref[...]
    col = lax.broadcasted_iota(jnp.int32, x.shape, 1)
    z = jnp.where(col == 0, x, -x)
    ls = jnp.minimum(z, 0.0) - jnp.log1p(jnp.exp(-jnp.abs(z)))
    o_ref[0, 0] = -jnp.sum(ls) / jnp.float32(BATCH)


def kernel(center, context, W_center, W_context):
    dk = jax.random.key(123)
    keep = jax.random.bernoulli(jax.random.fold_in(dk, 0), 0.9, (BATCH, EMB))
    mask_scale = jnp.where(keep, jnp.float32(1.0 / 0.9), jnp.float32(0.0))
    noise_idx = jax.random.randint(jax.random.fold_in(dk, 1), (BATCH * NEGS,),
                                   0, VOCAB).astype(jnp.int32)
    noise3 = noise_idx.reshape(NW, G, NEG_G)

    dots = _sc_dots(center.astype(jnp.int32), context.astype(jnp.int32),
                    noise3, mask_scale, W_center, W_context)

    loss = pl.pallas_call(
        _tc_loss_kernel,
        out_shape=jax.ShapeDtypeStruct((1, 1), jnp.float32),
        out_specs=pl.BlockSpec(memory_space=pltpu.SMEM),
    )(dots)
    return loss[0, 0]


# speedup vs baseline: 1.6739x; 1.4781x over previous
import functools

import jax
import jax.numpy as jnp
from jax import lax
from jax.experimental import pallas as pl
from jax.experimental.pallas import tpu as pltpu
from jax.experimental.pallas import tpu_sc as plsc

VOCAB = 1000000
EMB = 64
BATCH = 16384
NEGS = 15

NC = 2
NS = 16
L = 16
NW = NC * NS
BPW = BATCH // NW
GSZ = 16
G = BPW // GSZ
NSLOT = 1 + NEGS
NEG_G = NSLOT * GSZ
NCH = EMB // L
NPW = BPW * NSLOT
PR = 2 * EMB


def _sc_dots_kernel(center_hbm, context_hbm, noise_hbm, mask_hbm,
                    wc_hbm, wx_hbm, out_hbm,
                    cst_v, xst_v, nst_v, cfi_v, xfi_v, nfi_v,
                    cen0, ctx0, msk0, neg0, cen1, ctx1, msk1, neg1,
                    dots_v, sem0, sem1):
    wid = lax.axis_index("s") * NC + lax.axis_index("c")
    base = pl.multiple_of(wid * BPW, BPW)

    pltpu.sync_copy(center_hbm.at[wid], cst_v)
    pltpu.sync_copy(context_hbm.at[wid], xst_v)
    pltpu.sync_copy(noise_hbm.at[wid], nst_v)

    def c_tr(i, carry):
        r = i >> 3
        c = pl.multiple_of((i & 7) * L, L)
        f = pl.multiple_of(i * L, L)
        cfi_v[pl.ds(f, L)] = cst_v[r, pl.ds(c, L)]
        xfi_v[pl.ds(f, L)] = xst_v[r, pl.ds(c, L)]
        return carry

    lax.fori_loop(0, BPW // L, c_tr, 0)

    def n_tr(i, carry):
        r = i >> 3
        c = pl.multiple_of((i & 7) * L, L)
        f = pl.multiple_of(i * L, L)
        nfi_v[pl.ds(f, L)] = nst_v[r, pl.ds(c, L)]
        return carry

    lax.fori_loop(0, NPW // L, n_tr, 0)

    bufs = ((cen0, ctx0, msk0, neg0, sem0), (cen1, ctx1, msk1, neg1, sem1))

    def issue(g, s):
        cen_v, ctx_v, msk_v, neg_v, sem = bufs[s]
        row0 = pl.multiple_of(g * GSZ, GSZ)
        pltpu.async_copy(wc_hbm.at[cfi_v.at[pl.ds(row0, GSZ)]], cen_v, sem)
        pltpu.async_copy(wx_hbm.at[xfi_v.at[pl.ds(row0, GSZ)]], ctx_v, sem)
        pltpu.async_copy(wx_hbm.at[nfi_v.at[pl.ds(row0 * NSLOT, NEG_G)]],
                         neg_v, sem)
        mrow0 = pl.multiple_of((base + row0) // 2, GSZ // 2)
        pltpu.async_copy(mask_hbm.at[pl.ds(mrow0, GSZ // 2)], msk_v, sem)

    def drain(s):
        cen_v, ctx_v, msk_v, neg_v, sem = bufs[s]
        pltpu.make_async_copy(wc_hbm.at[pl.ds(0, GSZ)], cen_v, sem).wait()
        pltpu.make_async_copy(wx_hbm.at[pl.ds(0, GSZ)], ctx_v, sem).wait()
        pltpu.make_async_copy(wx_hbm.at[pl.ds(0, NEG_G)], neg_v, sem).wait()
        pltpu.make_async_copy(mask_hbm.at[pl.ds(0, GSZ // 2)], msk_v,
                              sem).wait()

    lane = lax.iota(jnp.int32, L)
    lane_mask = [lane == k for k in range(L)]

    def compute(g, s):
        cen_v, ctx_v, msk_v, neg_v, _ = bufs[s]
        row0 = pl.multiple_of(g * GSZ, GSZ)

        def body(b, carry):
            q = row0 + b
            hm = pl.multiple_of((b & 1) * EMB, EMB)
            cm = [cen_v[b, pl.ds(c * L, L)]
                  * msk_v[b >> 1, pl.ds(hm + c * L, L)]
                  for c in range(NCH)]
            t = cm[0] * ctx_v[b, pl.ds(0, L)]
            for c in range(1, NCH):
                t = t + cm[c] * ctx_v[b, pl.ds(c * L, L)]
            acc = jnp.where(lane_mask[0], jnp.sum(t), 0.0)
            for k in range(1, NSLOT):
                nrow = b * NSLOT + k
                t = cm[0] * neg_v[nrow, pl.ds(0, L)]
                for c in range(1, NCH):
                    t = t + cm[c] * neg_v[nrow, pl.ds(c * L, L)]
                acc = jnp.where(lane_mask[k], jnp.sum(t), acc)
            f = q * L
            dots_v[f >> 7, pl.ds(pl.multiple_of(f & 127, L), L)] = acc
            return carry

        lax.fori_loop(0, GSZ, body, 0)

    issue(0, 0)

    def pair(i, carry):
        g0 = pl.multiple_of(i * 2, 2)
        issue(g0 + 1, 1)
        drain(0)
        compute(g0, 0)

        @pl.when(g0 + 2 < G)
        def _():
            issue(g0 + 2, 0)

        drain(1)
        compute(g0 + 1, 1)
        return carry

    lax.fori_loop(0, G // 2, pair, 0)
    pltpu.sync_copy(dots_v, out_hbm.at[wid])


@functools.partial(
    pl.kernel,
    out_type=jax.ShapeDtypeStruct((NW, BPW * NSLOT // PR, PR), jnp.float32),
    mesh=plsc.VectorSubcoreMesh(core_axis_name="c", subcore_axis_name="s"),
    compiler_params=pltpu.CompilerParams(needs_layout_passes=False,
                                         use_tc_tiling_on_sc=True),
    scratch_types=[
        pltpu.VMEM((BPW // PR, PR), jnp.int32),
        pltpu.VMEM((BPW // PR, PR), jnp.int32),
        pltpu.VMEM((NPW // PR, PR), jnp.int32),
        pltpu.VMEM((BPW,), jnp.int32),
        pltpu.VMEM((BPW,), jnp.int32),
        pltpu.VMEM((NPW,), jnp.int32),
        pltpu.VMEM((GSZ, PR), jnp.float32),
        pltpu.VMEM((GSZ, PR), jnp.float32),
        pltpu.VMEM((GSZ // 2, PR), jnp.float32),
        pltpu.VMEM((NEG_G, PR), jnp.float32),
        pltpu.VMEM((GSZ, PR), jnp.float32),
        pltpu.VMEM((GSZ, PR), jnp.float32),
        pltpu.VMEM((GSZ // 2, PR), jnp.float32),
        pltpu.VMEM((NEG_G, PR), jnp.float32),
        pltpu.VMEM((BPW * NSLOT // PR, PR), jnp.float32),
        pltpu.SemaphoreType.DMA,
        pltpu.SemaphoreType.DMA,
    ],
)
def _sc_dots(center, context, noise, mask, wc, wx, out,
             cst_v, xst_v, nst_v, cfi_v, xfi_v, nfi_v,
             cen0, ctx0, msk0, neg0, cen1, ctx1, msk1, neg1,
             dots_v, sem0, sem1):
    _sc_dots_kernel(center, context, noise, mask, wc, wx, out,
                    cst_v, xst_v, nst_v, cfi_v, xfi_v, nfi_v,
                    cen0, ctx0, msk0, neg0, cen1, ctx1, msk1, neg1,
                    dots_v, sem0, sem1)


VB = 8192


def _tc_relayout_kernel(wt_ref, o_ref):
    t = jnp.transpose(wt_ref[...], (1, 0))
    o_ref[...] = jnp.concatenate([t, t], axis=1)


def _tc_relayout(wt):
    return pl.pallas_call(
        _tc_relayout_kernel,
        grid=(VOCAB // VB,),
        in_specs=[pl.BlockSpec((EMB, VB), lambda j: (0, j))],
        out_specs=pl.BlockSpec((VB, PR), lambda j: (j, 0)),
        out_shape=jax.ShapeDtypeStruct((VOCAB, PR), jnp.float32),
    )(wt)


def _tc_loss_kernel(dots_ref, o_ref):
    x = dots_ref[...]
    col = lax.broadcasted_iota(jnp.int32, x.shape, 2)
    z = jnp.where((col & (L - 1)) == 0, x, -x)
    ls = jnp.minimum(z, 0.0) - jnp.log1p(jnp.exp(-jnp.abs(z)))
    o_ref[0, 0] = -jnp.sum(ls) / jnp.float32(BATCH)


def kernel(center, context, W_center, W_context):
    dk = jax.random.key(123)
    keep = jax.random.bernoulli(jax.random.fold_in(dk, 0), 0.9, (BATCH, EMB))
    mask_scale = jnp.where(keep, jnp.float32(1.0 / 0.9), jnp.float32(0.0))
    noise_idx = jax.random.randint(jax.random.fold_in(dk, 1), (BATCH * NEGS,),
                                   0, VOCAB).astype(jnp.int32)
    noise15 = noise_idx.reshape(BATCH, NEGS)
    noise16 = jnp.concatenate([noise15[:, :1], noise15], axis=1)

    dots = _sc_dots(
        center.astype(jnp.int32).reshape(NW, BPW // PR, PR),
        context.astype(jnp.int32).reshape(NW, BPW // PR, PR),
        noise16.reshape(NW, NPW // PR, PR),
        mask_scale.reshape(BATCH * EMB // PR, PR),
        _tc_relayout(W_center.T),
        _tc_relayout(W_context.T),
    )

    loss = pl.pallas_call(
        _tc_loss_kernel,
        out_shape=jax.ShapeDtypeStruct((1, 1), jnp.float32),
        out_specs=pl.BlockSpec(memory_space=pltpu.SMEM),
    )(dots)
    return loss[0, 0]


# speedup vs baseline: 1.8105x; 1.0816x over previous
import functools

import jax
import jax.numpy as jnp
from jax import lax
from jax.experimental import pallas as pl
from jax.experimental.pallas import tpu as pltpu
from jax.experimental.pallas import tpu_sc as plsc

VOCAB = 1000000
EMB = 64
BATCH = 16384
NEGS = 15

NC = 2
NS = 16
L = 16
NW = NC * NS
BPW = BATCH // NW
GSZ = 16
G = BPW // GSZ
NSLOT = 1 + NEGS
NEG_G = NSLOT * GSZ
NCH = EMB // L
NPW = BPW * NSLOT
PR = 2 * EMB


def _sc_dots_kernel(center_hbm, context_hbm, noise_hbm, mask_hbm,
                    wc_hbm, wx_hbm, out_hbm,
                    cst_v, xst_v, nst_v, cfi_v, xfi_v, nfi_v,
                    cen0, ctx0, msk0, neg0, cen1, ctx1, msk1, neg1,
                    dots_v, sem0, sem1):
    wid = lax.axis_index("s") * NC + lax.axis_index("c")
    base = pl.multiple_of(wid * BPW, BPW)

    pltpu.sync_copy(center_hbm.at[wid], cst_v)
    pltpu.sync_copy(context_hbm.at[wid], xst_v)
    pltpu.sync_copy(noise_hbm.at[wid], nst_v)

    def c_tr(i, carry):
        r = i >> 3
        c = pl.multiple_of((i & 7) * L, L)
        f = pl.multiple_of(i * L, L)
        cfi_v[pl.ds(f, L)] = cst_v[r, pl.ds(c, L)]
        xfi_v[pl.ds(f, L)] = xst_v[r, pl.ds(c, L)]
        return carry

    lax.fori_loop(0, BPW // L, c_tr, 0)

    def n_tr(i, carry):
        r = i >> 3
        c = pl.multiple_of((i & 7) * L, L)
        f = pl.multiple_of(i * L, L)
        nfi_v[pl.ds(f, L)] = nst_v[r, pl.ds(c, L)]
        return carry

    lax.fori_loop(0, NPW // L, n_tr, 0)

    bufs = ((cen0, ctx0, msk0, neg0, sem0), (cen1, ctx1, msk1, neg1, sem1))

    def issue(g, s):
        cen_v, ctx_v, msk_v, neg_v, sem = bufs[s]
        row0 = pl.multiple_of(g * GSZ, GSZ)
        pltpu.async_copy(wc_hbm.at[cfi_v.at[pl.ds(row0, GSZ)]], cen_v, sem)
        pltpu.async_copy(wx_hbm.at[xfi_v.at[pl.ds(row0, GSZ)]], ctx_v, sem)
        pltpu.async_copy(wx_hbm.at[nfi_v.at[pl.ds(row0 * NSLOT, NEG_G)]],
                         neg_v, sem)
        mrow0 = pl.multiple_of((base + row0) // 2, GSZ // 2)
        pltpu.async_copy(mask_hbm.at[pl.ds(mrow0, GSZ // 2)], msk_v, sem)

    def drain(s):
        cen_v, ctx_v, msk_v, neg_v, sem = bufs[s]
        pltpu.make_async_copy(wc_hbm.at[pl.ds(0, GSZ)], cen_v, sem).wait()
        pltpu.make_async_copy(wx_hbm.at[pl.ds(0, GSZ)], ctx_v, sem).wait()
        pltpu.make_async_copy(wx_hbm.at[pl.ds(0, NEG_G)], neg_v, sem).wait()
        pltpu.make_async_copy(mask_hbm.at[pl.ds(0, GSZ // 2)], msk_v,
                              sem).wait()

    lane = lax.iota(jnp.int32, L)
    lane_mask = [lane == k for k in range(L)]

    def compute(g, s):
        cen_v, ctx_v, msk_v, neg_v, _ = bufs[s]
        row0 = pl.multiple_of(g * GSZ, GSZ)

        def body(b, carry):
            q = row0 + b
            hm = pl.multiple_of((b & 1) * EMB, EMB)
            cm = [cen_v[b, pl.ds(c * L, L)]
                  * msk_v[b >> 1, pl.ds(hm + c * L, L)]
                  for c in range(NCH)]
            t = cm[0] * ctx_v[b, pl.ds(0, L)]
            for c in range(1, NCH):
                t = t + cm[c] * ctx_v[b, pl.ds(c * L, L)]
            acc = jnp.where(lane_mask[0], jnp.sum(t), 0.0)
            for k in range(1, NSLOT):
                nrow = b * NSLOT + k
                t = cm[0] * neg_v[nrow, pl.ds(0, L)]
                for c in range(1, NCH):
                    t = t + cm[c] * neg_v[nrow, pl.ds(c * L, L)]
                acc = jnp.where(lane_mask[k], jnp.sum(t), acc)
            f = q * L
            dots_v[f >> 7, pl.ds(pl.multiple_of(f & 127, L), L)] = acc
            return carry

        lax.fori_loop(0, GSZ, body, 0)

    issue(0, 0)

    def pair(i, carry):
        g0 = pl.multiple_of(i * 2, 2)
        issue(g0 + 1, 1)
        drain(0)
        compute(g0, 0)

        @pl.when(g0 + 2 < G)
        def _():
            issue(g0 + 2, 0)

        drain(1)
        compute(g0 + 1, 1)
        return carry

    lax.fori_loop(0, G // 2, pair, 0)
    pltpu.sync_copy(dots_v, out_hbm.at[wid])


@functools.partial(
    pl.kernel,
    out_type=jax.ShapeDtypeStruct((NW, BPW * NSLOT // PR, PR), jnp.float32),
    mesh=plsc.VectorSubcoreMesh(core_axis_name="c", subcore_axis_name="s"),
    compiler_params=pltpu.CompilerParams(needs_layout_passes=False,
                                         use_tc_tiling_on_sc=True),
    scratch_types=[
        pltpu.VMEM((BPW // PR, PR), jnp.int32),
        pltpu.VMEM((BPW // PR, PR), jnp.int32),
        pltpu.VMEM((NPW // PR, PR), jnp.int32),
        pltpu.VMEM((BPW,), jnp.int32),
        pltpu.VMEM((BPW,), jnp.int32),
        pltpu.VMEM((NPW,), jnp.int32),
        pltpu.VMEM((GSZ, PR), jnp.float32),
        pltpu.VMEM((GSZ, PR), jnp.float32),
        pltpu.VMEM((GSZ // 2, PR), jnp.float32),
        pltpu.VMEM((NEG_G, PR), jnp.float32),
        pltpu.VMEM((GSZ, PR), jnp.float32),
        pltpu.VMEM((GSZ, PR), jnp.float32),
        pltpu.VMEM((GSZ // 2, PR), jnp.float32),
        pltpu.VMEM((NEG_G, PR), jnp.float32),
        pltpu.VMEM((BPW * NSLOT // PR, PR), jnp.float32),
        pltpu.SemaphoreType.DMA,
        pltpu.SemaphoreType.DMA,
    ],
)
def _sc_dots(center, context, noise, mask, wc, wx, out,
             cst_v, xst_v, nst_v, cfi_v, xfi_v, nfi_v,
             cen0, ctx0, msk0, neg0, cen1, ctx1, msk1, neg1,
             dots_v, sem0, sem1):
    _sc_dots_kernel(center, context, noise, mask, wc, wx, out,
                    cst_v, xst_v, nst_v, cfi_v, xfi_v, nfi_v,
                    cen0, ctx0, msk0, neg0, cen1, ctx1, msk1, neg1,
                    dots_v, sem0, sem1)


VB = 16384


def _tc_relayout_kernel(wt_ref, o_ref):
    t = jnp.transpose(wt_ref[...], (1, 0))
    o_ref[...] = jnp.concatenate([t, t], axis=1)


def _tc_relayout(wt):
    return pl.pallas_call(
        _tc_relayout_kernel,
        grid=(pl.cdiv(VOCAB, VB),),
        in_specs=[pl.BlockSpec((EMB, VB), lambda j: (0, j))],
        out_specs=pl.BlockSpec((VB, PR), lambda j: (j, 0)),
        out_shape=jax.ShapeDtypeStruct((VOCAB, PR), jnp.float32),
    )(wt)


def _tc_loss_kernel(dots_ref, o_ref):
    x = dots_ref[...]
    col = lax.broadcasted_iota(jnp.int32, x.shape, 2)
    z = jnp.where((col & (L - 1)) == 0, x, -x)
    ls = jnp.minimum(z, 0.0) - jnp.log1p(jnp.exp(-jnp.abs(z)))
    o_ref[0, 0] = -jnp.sum(ls) / jnp.float32(BATCH)


def kernel(center, context, W_center, W_context):
    dk = jax.random.key(123)
    keep = jax.random.bernoulli(jax.random.fold_in(dk, 0), 0.9, (BATCH, EMB))
    mask_scale = jnp.where(keep, jnp.float32(1.0 / 0.9), jnp.float32(0.0))
    noise_idx = jax.random.randint(jax.random.fold_in(dk, 1), (BATCH * NEGS,),
                                   0, VOCAB).astype(jnp.int32)
    noise15 = noise_idx.reshape(BATCH, NEGS)
    noise16 = jnp.concatenate([noise15[:, :1], noise15], axis=1)

    dots = _sc_dots(
        center.astype(jnp.int32).reshape(NW, BPW // PR, PR),
        context.astype(jnp.int32).reshape(NW, BPW // PR, PR),
        noise16.reshape(NW, NPW // PR, PR),
        mask_scale.reshape(BATCH * EMB // PR, PR),
        _tc_relayout(W_center.T),
        _tc_relayout(W_context.T),
    )

    loss = pl.pallas_call(
        _tc_loss_kernel,
        out_shape=jax.ShapeDtypeStruct((1, 1), jnp.float32),
        out_specs=pl.BlockSpec(memory_space=pltpu.SMEM),
    )(dots)
    return loss[0, 0]
